# baseline (speedup 1.0000x reference)
import functools

import jax
import jax.numpy as jnp
from jax import lax
from jax.experimental import pallas as pl
from jax.experimental.pallas import tpu as pltpu
from jax.experimental.pallas import tpu_sc as plsc

TOPK = 20
N = 4096
CHUNK = 16
NCHUNK = N // CHUNK
BR = 256

NB = 8
ND = 64
NW = 32
ROWS_PER_W = N // NW
GB = 16
NBATCH = ROWS_PER_W // GB
IDX_PER_STREAM = 80


def _cand_block(m_ref, p_ref, src1_ref, src2_ref, gidx_ref, s1t_ref, s2t_ref):
    s1t_ref[...] = jnp.concatenate([src1_ref[b] for b in range(NB)], axis=1)
    s2t_ref[...] = jnp.concatenate([src2_ref[b] for b in range(NB)], axis=1)
    a = m_ref[...]
    wm = a
    for s in (1, 2, 4, 8):
        wm = jnp.maximum(wm, pltpu.roll(wm, N - s, axis=1))
    cm = jnp.dot(wm, p_ref[...], preferred_element_type=jnp.float32)
    ci = lax.bitcast_convert_type(cm, jnp.int32)
    s = ci ^ ((ci >> 31) & jnp.int32(0x7FFFFFFF))
    iota = lax.broadcasted_iota(jnp.int32, (BR, NCHUNK), 1)
    k = (s & jnp.int32(~0xFF)) | (jnp.int32(NCHUNK - 1) - iota)
    ids = []
    for _ in range(TOPK):
        m = jnp.max(k, axis=1, keepdims=True)
        ids.append(jnp.int32(NCHUNK - 1) - (m & jnp.int32(0xFF)))
        k = jnp.where(k == m, jnp.int32(-2**31), k)
    cids = jnp.concatenate(ids, axis=1)
    row = pl.program_id(0) * BR + lax.broadcasted_iota(
        jnp.int32, (BR, TOPK), 0)
    gidx_ref[...] = row * NCHUNK + cids


def _candidates(m, src1, src2):
    p = (jnp.arange(N, dtype=jnp.int32)[:, None]
         == CHUNK * jnp.arange(NCHUNK, dtype=jnp.int32)[None, :]
         ).astype(jnp.float32)
    bd = NB * ND
    return pl.pallas_call(
        _cand_block,
        grid=(N // BR,),
        in_specs=[
            pl.BlockSpec((BR, N), lambda i: (i, 0)),
            pl.BlockSpec((N, NCHUNK), lambda i: (0, 0)),
            pl.BlockSpec((NB, BR, ND), lambda i: (0, i, 0)),
            pl.BlockSpec((NB, BR, ND), lambda i: (0, i, 0)),
        ],
        out_specs=[
            pl.BlockSpec((BR, TOPK), lambda i: (i, 0)),
            pl.BlockSpec((BR, bd), lambda i: (i, 0)),
            pl.BlockSpec((BR, bd), lambda i: (i, 0)),
        ],
        out_shape=[
            jax.ShapeDtypeStruct((N, TOPK), jnp.int32),
            jax.ShapeDtypeStruct((N, bd), jnp.float32),
            jax.ShapeDtypeStruct((N, bd), jnp.float32),
        ],
    )(m, p, src1, src2)


def _sortd(x):
    k, _ = plsc.sort_key_val(x, x, descending=True)
    return k


def _rev(x):
    return lax.rev(x, (0,))


def _gather16(x, idx):
    return lax.gather(
        x, idx[:, None],
        lax.GatherDimensionNumbers(
            offset_dims=(), collapsed_slice_dims=(0,), start_index_map=(0,)),
        (1,), mode=lax.GatherScatterMode.PROMISE_IN_BOUNDS)


def _merge_pair(hi, lo):
    rlo = _rev(lo)
    top = jnp.maximum(hi, rlo)
    bot = jnp.minimum(hi, rlo)
    return _sortd(top), _sortd(bot)


def _sc_t20(m16, gidx_flat):
    mesh = plsc.VectorSubcoreMesh(core_axis_name="c", subcore_axis_name="s")

    @functools.partial(
        pl.kernel,
        mesh=mesh,
        out_type=jax.ShapeDtypeStruct((N,), jnp.float32),
        compiler_params=pltpu.CompilerParams(
            needs_layout_passes=False, use_tc_tiling_on_sc=False),
        scratch_types=[
            pltpu.VMEM((ROWS_PER_W * TOPK,), jnp.int32),
            pltpu.VMEM((GB * TOPK, CHUNK), jnp.float32),
            pltpu.VMEM((GB * TOPK, CHUNK), jnp.float32),
            pltpu.VMEM((16,), jnp.float32),
            pltpu.SemaphoreType.DMA,
            pltpu.SemaphoreType.DMA,
        ],
    )
    def k(m16_hbm, gidx_hbm, out_hbm, gidx_v, cand0, cand1, t20_v, sem0, sem1):
        wid = lax.axis_index("s") * 2 + lax.axis_index("c")
        row0 = wid * ROWS_PER_W
        pltpu.sync_copy(
            gidx_hbm.at[pl.ds(row0 * TOPK, ROWS_PER_W * TOPK)], gidx_v)

        cands = (cand0, cand1)
        sems = (sem0, sem1)

        def descriptors(b, half):
            buf, sem = cands[half], sems[half]
            ds = []
            for i in range(4):
                idx = gidx_v.at[pl.ds(b * GB * TOPK + i * IDX_PER_STREAM,
                                      IDX_PER_STREAM)]
                ds.append(pltpu.make_async_copy(
                    m16_hbm.at[idx],
                    buf.at[pl.ds(i * IDX_PER_STREAM, IDX_PER_STREAM), :],
                    sem))
            return ds

        def fire(b, half):
            for dsc in descriptors(b, half):
                dsc.start()

        lane = lax.iota(jnp.int32, 16)
        sel3 = jnp.full((16,), 3, jnp.int32)

        fire(0, 0)
        fire(1, 1)

        def process(b, half):
            buf = cands[half]
            for dsc in descriptors(b, half):
                dsc.wait()

            def row_body(g, acc):
                def cvec(j):
                    return buf[g * TOPK + j]

                a0 = _sortd(cvec(0))
                a1 = _sortd(cvec(1))
                a0, a1 = _merge_pair(a0, a1)
                for j in range(2, TOPK):
                    bsrt = _sortd(cvec(j))
                    a1 = _sortd(jnp.maximum(a1, _rev(bsrt)))
                    a0, a1 = _merge_pair(a0, a1)
                t20 = _gather16(a1, sel3)
                return jnp.where(lane == g, t20, acc)

            acc = lax.fori_loop(0, GB, row_body, jnp.zeros((16,), jnp.float32))
            t20_v[...] = acc

            @pl.when(b + 2 < NBATCH)
            def _():
                fire(b + 2, half)

            pltpu.sync_copy(t20_v, out_hbm.at[pl.ds(row0 + b * GB, GB)])

        def super_body(sb, carry):
            process(2 * sb, 0)
            process(2 * sb + 1, 1)
            return carry

        lax.fori_loop(0, NBATCH // 2, super_body, 0)

    return k(m16, gidx_flat)


def _out_block(m_ref, src2t_ref, src1t_ref, t20_ref, out_ref):
    a = m_ref[...]
    rowmax = jnp.max(a, axis=1, keepdims=True)
    e = jnp.exp(a - rowmax)
    denom = jnp.sum(e, axis=1, keepdims=True)
    w = jnp.where(a >= t20_ref[...], e / denom, 0.0)
    out_ref[...] = (
        jnp.dot(w, src2t_ref[...], preferred_element_type=jnp.float32)
        + src1t_ref[...]
    )


def _masked_matmul(m, src2t, src1t, t20col):
    bd = src2t.shape[1]
    return pl.pallas_call(
        _out_block,
        grid=(N // BR,),
        in_specs=[
            pl.BlockSpec((BR, N), lambda i: (i, 0)),
            pl.BlockSpec((N, bd), lambda i: (0, 0)),
            pl.BlockSpec((BR, bd), lambda i: (i, 0)),
            pl.BlockSpec((BR, 1), lambda i: (i, 0)),
        ],
        out_specs=pl.BlockSpec((BR, bd), lambda i: (i, 0)),
        out_shape=jax.ShapeDtypeStruct((N, bd), jnp.float32),
    )(m, src2t, src1t, t20col)


def kernel(src1, src2, memoryMartix):
    B, n, d = src1.shape
    gidx, src1t, src2t = _candidates(memoryMartix, src1, src2)
    t20 = _sc_t20(
        memoryMartix.reshape(N * NCHUNK, CHUNK), gidx.reshape(-1))
    out_t = _masked_matmul(memoryMartix, src2t, src1t, t20.reshape(n, 1))
    return out_t.reshape(n, B, d).transpose(1, 0, 2)


# speedup vs baseline: 1.2630x; 1.2630x over previous
import functools

import jax
import jax.numpy as jnp
from jax import lax
from jax.experimental import pallas as pl
from jax.experimental.pallas import tpu as pltpu
from jax.experimental.pallas import tpu_sc as plsc

TOPK = 20
N = 4096
CHUNK = 16
NCHUNK = N // CHUNK
BR = 256

NB = 8
ND = 64
NW = 32
ROWS_PER_W = N // NW
GB = 16
NBATCH = ROWS_PER_W // GB
IDX_PER_STREAM = 80


def _cand_block(m_ref, p_ref, gidx_ref):
    a = m_ref[...]
    wm = a
    for s in (1, 2, 4, 8):
        wm = jnp.maximum(wm, pltpu.roll(wm, N - s, axis=1))
    cm = jnp.dot(wm, p_ref[...], preferred_element_type=jnp.float32)
    ci = lax.bitcast_convert_type(cm, jnp.int32)
    s = ci ^ ((ci >> 31) & jnp.int32(0x7FFFFFFF))
    iota = lax.broadcasted_iota(jnp.int32, (BR, NCHUNK), 1)
    k = (s & jnp.int32(~0xFF)) | (jnp.int32(NCHUNK - 1) - iota)
    ids = []
    for _ in range(TOPK):
        m = jnp.max(k, axis=1, keepdims=True)
        ids.append(jnp.int32(NCHUNK - 1) - (m & jnp.int32(0xFF)))
        k = jnp.where(k == m, jnp.int32(-2**31), k)
    cids = jnp.concatenate(ids, axis=1)
    row = pl.program_id(0) * BR + lax.broadcasted_iota(
        jnp.int32, (BR, TOPK), 0)
    gidx_ref[...] = row * NCHUNK + cids


def _candidates(m):
    p = (jnp.arange(N, dtype=jnp.int32)[:, None]
         == CHUNK * jnp.arange(NCHUNK, dtype=jnp.int32)[None, :]
         ).astype(jnp.float32)
    return pl.pallas_call(
        _cand_block,
        grid=(N // BR,),
        in_specs=[
            pl.BlockSpec((BR, N), lambda i: (i, 0)),
            pl.BlockSpec((N, NCHUNK), lambda i: (0, 0)),
        ],
        out_specs=pl.BlockSpec((BR, TOPK), lambda i: (i, 0)),
        out_shape=jax.ShapeDtypeStruct((N, TOPK), jnp.int32),
    )(m, p)


def _sortd(x):
    k, _ = plsc.sort_key_val(x, x, descending=True)
    return k


def _rev(x):
    return lax.rev(x, (0,))


def _gather16(x, idx):
    return lax.gather(
        x, idx[:, None],
        lax.GatherDimensionNumbers(
            offset_dims=(), collapsed_slice_dims=(0,), start_index_map=(0,)),
        (1,), mode=lax.GatherScatterMode.PROMISE_IN_BOUNDS)


def _merge_pair(hi, lo):
    rlo = _rev(lo)
    top = jnp.maximum(hi, rlo)
    bot = jnp.minimum(hi, rlo)
    return _sortd(top), _sortd(bot)


def _sc_t20(m16, gidx_flat):
    mesh = plsc.VectorSubcoreMesh(core_axis_name="c", subcore_axis_name="s")

    @functools.partial(
        pl.kernel,
        mesh=mesh,
        out_type=jax.ShapeDtypeStruct((N,), jnp.float32),
        compiler_params=pltpu.CompilerParams(
            needs_layout_passes=False, use_tc_tiling_on_sc=False),
        scratch_types=[
            pltpu.VMEM((ROWS_PER_W * TOPK,), jnp.int32),
            pltpu.VMEM((GB * TOPK, CHUNK), jnp.float32),
            pltpu.VMEM((GB * TOPK, CHUNK), jnp.float32),
            pltpu.VMEM((16,), jnp.float32),
            pltpu.SemaphoreType.DMA,
            pltpu.SemaphoreType.DMA,
        ],
    )
    def k(m16_hbm, gidx_hbm, out_hbm, gidx_v, cand0, cand1, t20_v, sem0, sem1):
        wid = lax.axis_index("s") * 2 + lax.axis_index("c")
        row0 = wid * ROWS_PER_W
        pltpu.sync_copy(
            gidx_hbm.at[pl.ds(row0 * TOPK, ROWS_PER_W * TOPK)], gidx_v)

        cands = (cand0, cand1)
        sems = (sem0, sem1)

        def descriptors(b, half):
            buf, sem = cands[half], sems[half]
            ds = []
            for i in range(4):
                idx = gidx_v.at[pl.ds(b * GB * TOPK + i * IDX_PER_STREAM,
                                      IDX_PER_STREAM)]
                ds.append(pltpu.make_async_copy(
                    m16_hbm.at[idx],
                    buf.at[pl.ds(i * IDX_PER_STREAM, IDX_PER_STREAM), :],
                    sem))
            return ds

        def fire(b, half):
            for dsc in descriptors(b, half):
                dsc.start()

        lane = lax.iota(jnp.int32, 16)
        sel3 = jnp.full((16,), 3, jnp.int32)

        fire(0, 0)
        fire(1, 1)

        def process(b, half):
            buf = cands[half]
            for dsc in descriptors(b, half):
                dsc.wait()

            def one_row(g):
                def cvec(j):
                    return buf[g * TOPK + j]

                a0 = _sortd(cvec(0))
                a1 = _sortd(cvec(1))
                a0, a1 = _merge_pair(a0, a1)
                for j in range(2, TOPK):
                    bsrt = _sortd(cvec(j))
                    a1 = _sortd(jnp.maximum(a1, _rev(bsrt)))
                    a0, a1 = _merge_pair(a0, a1)
                return _gather16(a1, sel3)

            def row_body(g2, acc):
                ta = one_row(2 * g2)
                tb = one_row(2 * g2 + 1)
                acc = jnp.where(lane == 2 * g2, ta, acc)
                return jnp.where(lane == 2 * g2 + 1, tb, acc)

            acc = lax.fori_loop(0, GB // 2, row_body,
                                jnp.zeros((16,), jnp.float32))
            t20_v[...] = acc

            @pl.when(b + 2 < NBATCH)
            def _():
                fire(b + 2, half)

            pltpu.sync_copy(t20_v, out_hbm.at[pl.ds(row0 + b * GB, GB)])

        def super_body(sb, carry):
            process(2 * sb, 0)
            process(2 * sb + 1, 1)
            return carry

        lax.fori_loop(0, NBATCH // 2, super_body, 0)

    return k(m16, gidx_flat)


def _out_block(m_ref, src2t_ref, src1t_ref, t20_ref, out_ref):
    a = m_ref[...]
    rowmax = jnp.max(a, axis=1, keepdims=True)
    e = jnp.exp(a - rowmax)
    denom = jnp.sum(e, axis=1, keepdims=True)
    w = jnp.where(a >= t20_ref[...], e / denom, 0.0)
    out_ref[...] = (
        jnp.dot(w, src2t_ref[...], preferred_element_type=jnp.float32)
        + src1t_ref[...]
    )


def _masked_matmul(m, src2t, src1t, t20col):
    bd = src2t.shape[1]
    return pl.pallas_call(
        _out_block,
        grid=(N // BR,),
        in_specs=[
            pl.BlockSpec((BR, N), lambda i: (i, 0)),
            pl.BlockSpec((N, bd), lambda i: (0, 0)),
            pl.BlockSpec((BR, bd), lambda i: (i, 0)),
            pl.BlockSpec((BR, 1), lambda i: (i, 0)),
        ],
        out_specs=pl.BlockSpec((BR, bd), lambda i: (i, 0)),
        out_shape=jax.ShapeDtypeStruct((N, bd), jnp.float32),
    )(m, src2t, src1t, t20col)


def kernel(src1, src2, memoryMartix):
    B, n, d = src1.shape
    bd = B * d
    src2t = src2.transpose(1, 0, 2).reshape(n, bd)
    src1t = src1.transpose(1, 0, 2).reshape(n, bd)
    gidx = _candidates(memoryMartix)
    t20 = _sc_t20(
        memoryMartix.reshape(N * NCHUNK, CHUNK), gidx.reshape(-1))
    out_t = _masked_matmul(memoryMartix, src2t, src1t, t20.reshape(n, 1))
    return out_t.reshape(n, B, d).transpose(1, 0, 2)


# speedup vs baseline: 1.4081x; 1.1149x over previous
import functools

import jax
import jax.numpy as jnp
from jax import lax
from jax.experimental import pallas as pl
from jax.experimental.pallas import tpu as pltpu
from jax.experimental.pallas import tpu_sc as plsc

TOPK = 20
N = 4096
CHUNK = 16
NCHUNK = N // CHUNK
BR = 512

NB = 8
ND = 64
NW = 32
ROWS_PER_W = N // NW
GB = 16
NBATCH = ROWS_PER_W // GB
IDX_PER_STREAM = 80


def _cand_block(m_ref, p_ref, gidx_ref):
    a = m_ref[...]
    wm = a
    for s in (1, 2, 4, 8):
        wm = jnp.maximum(wm, pltpu.roll(wm, N - s, axis=1))
    cm = jnp.dot(wm, p_ref[...], preferred_element_type=jnp.float32)
    ci = lax.bitcast_convert_type(cm, jnp.int32)
    s = ci ^ ((ci >> 31) & jnp.int32(0x7FFFFFFF))
    iota = lax.broadcasted_iota(jnp.int32, (BR, NCHUNK), 1)
    k = (s & jnp.int32(~0xFF)) | (jnp.int32(NCHUNK - 1) - iota)
    ids = []
    for _ in range(TOPK):
        m = jnp.max(k, axis=1, keepdims=True)
        ids.append(jnp.int32(NCHUNK - 1) - (m & jnp.int32(0xFF)))
        k = jnp.where(k == m, jnp.int32(-2**31), k)
    cids = jnp.concatenate(ids, axis=1)
    row = pl.program_id(0) * BR + lax.broadcasted_iota(
        jnp.int32, (BR, TOPK), 0)
    gidx_ref[...] = row * NCHUNK + cids


def _candidates(m):
    p = (jnp.arange(N, dtype=jnp.int32)[:, None]
         == CHUNK * jnp.arange(NCHUNK, dtype=jnp.int32)[None, :]
         ).astype(jnp.float32)
    return pl.pallas_call(
        _cand_block,
        grid=(N // BR,),
        in_specs=[
            pl.BlockSpec((BR, N), lambda i: (i, 0)),
            pl.BlockSpec((N, NCHUNK), lambda i: (0, 0)),
        ],
        out_specs=pl.BlockSpec((BR, TOPK), lambda i: (i, 0)),
        out_shape=jax.ShapeDtypeStruct((N, TOPK), jnp.int32),
    )(m, p)


def _sortd(x):
    k, _ = plsc.sort_key_val(x, x, descending=True)
    return k


def _rev(x):
    return lax.rev(x, (0,))


def _gather16(x, idx):
    return lax.gather(
        x, idx[:, None],
        lax.GatherDimensionNumbers(
            offset_dims=(), collapsed_slice_dims=(0,), start_index_map=(0,)),
        (1,), mode=lax.GatherScatterMode.PROMISE_IN_BOUNDS)


def _merge_pair(hi, lo):
    rlo = _rev(lo)
    top = jnp.maximum(hi, rlo)
    bot = jnp.minimum(hi, rlo)
    return _sortd(top), _sortd(bot)


def _sc_t20(m16, gidx_flat):
    mesh = plsc.VectorSubcoreMesh(core_axis_name="c", subcore_axis_name="s")

    @functools.partial(
        pl.kernel,
        mesh=mesh,
        out_type=jax.ShapeDtypeStruct((N,), jnp.float32),
        compiler_params=pltpu.CompilerParams(
            needs_layout_passes=False, use_tc_tiling_on_sc=False),
        scratch_types=[
            pltpu.VMEM((ROWS_PER_W * TOPK,), jnp.int32),
            pltpu.VMEM((GB * TOPK, CHUNK), jnp.float32),
            pltpu.VMEM((GB * TOPK, CHUNK), jnp.float32),
            pltpu.VMEM((16,), jnp.float32),
            pltpu.SemaphoreType.DMA,
            pltpu.SemaphoreType.DMA,
        ],
    )
    def k(m16_hbm, gidx_hbm, out_hbm, gidx_v, cand0, cand1, t20_v, sem0, sem1):
        wid = lax.axis_index("s") * 2 + lax.axis_index("c")
        row0 = wid * ROWS_PER_W
        pltpu.sync_copy(
            gidx_hbm.at[pl.ds(row0 * TOPK, ROWS_PER_W * TOPK)], gidx_v)

        cands = (cand0, cand1)
        sems = (sem0, sem1)

        def descriptors(b, half):
            buf, sem = cands[half], sems[half]
            ds = []
            for i in range(4):
                idx = gidx_v.at[pl.ds(b * GB * TOPK + i * IDX_PER_STREAM,
                                      IDX_PER_STREAM)]
                ds.append(pltpu.make_async_copy(
                    m16_hbm.at[idx],
                    buf.at[pl.ds(i * IDX_PER_STREAM, IDX_PER_STREAM), :],
                    sem))
            return ds

        def fire(b, half):
            for dsc in descriptors(b, half):
                dsc.start()

        lane = lax.iota(jnp.int32, 16)
        sel3 = jnp.full((16,), 3, jnp.int32)

        fire(0, 0)
        fire(1, 1)

        def process(b, half):
            buf = cands[half]
            for dsc in descriptors(b, half):
                dsc.wait()

            def one_row(g):
                def cvec(j):
                    return buf[g * TOPK + j]

                a0 = _sortd(cvec(0))
                a1 = _sortd(cvec(1))
                a0, a1 = _merge_pair(a0, a1)
                for j in range(2, TOPK):
                    bsrt = _sortd(cvec(j))
                    a1 = _sortd(jnp.maximum(a1, _rev(bsrt)))
                    a0, a1 = _merge_pair(a0, a1)
                return _gather16(a1, sel3)

            def row_body(g2, acc):
                ta = one_row(2 * g2)
                tb = one_row(2 * g2 + 1)
                acc = jnp.where(lane == 2 * g2, ta, acc)
                return jnp.where(lane == 2 * g2 + 1, tb, acc)

            acc = lax.fori_loop(0, GB // 2, row_body,
                                jnp.zeros((16,), jnp.float32))
            t20_v[...] = acc

            @pl.when(b + 2 < NBATCH)
            def _():
                fire(b + 2, half)

            pltpu.sync_copy(t20_v, out_hbm.at[pl.ds(row0 + b * GB, GB)])

        def super_body(sb, carry):
            process(2 * sb, 0)
            process(2 * sb + 1, 1)
            return carry

        lax.fori_loop(0, NBATCH // 2, super_body, 0)

    return k(m16, gidx_flat)


def _out_block(m_ref, src2t_ref, src1t_ref, t20_ref, out_ref):
    a = m_ref[...]
    rowmax = jnp.max(a, axis=1, keepdims=True)
    e = jnp.exp(a - rowmax)
    denom = jnp.sum(e, axis=1, keepdims=True)
    w = jnp.where(a >= t20_ref[...], e / denom, 0.0)
    out_ref[...] = (
        jnp.dot(w, src2t_ref[...], preferred_element_type=jnp.float32)
        + src1t_ref[...]
    )


def _masked_matmul(m, src2t, src1t, t20col):
    bd = src2t.shape[1]
    return pl.pallas_call(
        _out_block,
        grid=(N // BR,),
        in_specs=[
            pl.BlockSpec((BR, N), lambda i: (i, 0)),
            pl.BlockSpec((N, bd), lambda i: (0, 0)),
            pl.BlockSpec((BR, bd), lambda i: (i, 0)),
            pl.BlockSpec((BR, 1), lambda i: (i, 0)),
        ],
        out_specs=pl.BlockSpec((BR, bd), lambda i: (i, 0)),
        out_shape=jax.ShapeDtypeStruct((N, bd), jnp.float32),
    )(m, src2t, src1t, t20col)


def kernel(src1, src2, memoryMartix):
    B, n, d = src1.shape
    bd = B * d
    src2t = src2.transpose(1, 0, 2).reshape(n, bd)
    src1t = src1.transpose(1, 0, 2).reshape(n, bd)
    gidx = _candidates(memoryMartix)
    t20 = _sc_t20(
        memoryMartix.reshape(N * NCHUNK, CHUNK), gidx.reshape(-1))
    out_t = _masked_matmul(memoryMartix, src2t, src1t, t20.reshape(n, 1))
    return out_t.reshape(n, B, d).transpose(1, 0, 2)
